# baseline (speedup 1.0000x reference)
import functools

import jax
import jax.numpy as jnp
from jax import lax
from jax.experimental import pallas as pl
from jax.experimental.pallas import tpu as pltpu
from jax.experimental.pallas import tpu_sc as plsc

N = 320000
NUM_SEG = 10000
F1 = 128
G1 = 256
G2 = 128

NW = 32
C = N // NW
BLK = 1024
GRP = 16
NGRP = BLK // GRP
STG = 128
NEG = -3.0e38
NREG = F1 // 16
EMAX = N // BLK + NUM_SEG // STG + 4


def _sc_body(xs_hbm, ys_hbm, zs_hbm, cl_hbm, w1_hbm, b1_hbm,
             cprev_hbm, clast_hbm, out_hbm,
             xs_v, ys_v, zs_v, cl_v, w1_v, b1_v, cprev_v, clast_v, stg_v,
             acc_v):
    wid = lax.axis_index("s") * 2 + lax.axis_index("c")
    pltpu.sync_copy(w1_hbm, w1_v)
    pltpu.sync_copy(b1_hbm, b1_v)
    pltpu.sync_copy(cprev_hbm, cprev_v)
    pltpu.sync_copy(clast_hbm, clast_v)

    c_prev = cprev_v[pl.ds(wid, 16)][0]
    c_last = clast_v[pl.ds(wid, 16)][0]

    w = [[w1_v[r, pl.ds(16 * j, 16)] for j in range(NREG)] for r in range(3)]
    bias = [b1_v[pl.ds(16 * j, 16)] for j in range(NREG)]
    neg_vec = jnp.full((16,), NEG, jnp.float32)
    zero_vec = jnp.zeros((16,), jnp.float32)

    def zrow(g, _):
        for j in range(GRP):
            stg_v[pl.ds(g * (GRP * 16) + 16 * j, 16)] = neg_vec
        return 0
    lax.fori_loop(0, (STG * F1) // (GRP * 16), zrow, 0)
    for r in range(NREG):
        acc_v[pl.ds(16 * r, 16)] = neg_vec

    def event_body(_, est):
        (i, block_valid, paused, g_resume, done_data, stg_base, prev_id) = est
        done_all = (done_data == 1) & (stg_base > c_last)
        i_load = pl.multiple_of(jnp.minimum(i, N - BLK), 8)
        win_end = stg_base + STG
        lim = jnp.minimum(win_end, c_last + 1)

        need_load = (~done_all) & (done_data == 0) & (block_valid == 0)

        @pl.when(need_load)
        def _():
            pltpu.sync_copy(cl_hbm.at[pl.ds(i_load, BLK)], cl_v)
            pltpu.sync_copy(xs_hbm.at[pl.ds(i_load, BLK)], xs_v)
            pltpu.sync_copy(ys_hbm.at[pl.ds(i_load, BLK)], ys_v)
            pltpu.sync_copy(zs_hbm.at[pl.ds(i_load, BLK)], zs_v)

        block_valid = jnp.where(need_load, 1, block_valid)
        can_process = (~done_all) & (done_data == 0) & (block_valid == 1)

        def do_process(pst):
            (i, block_valid, paused, g_resume, done_data, stg_base,
             prev_id) = pst

            def group_fn(g, gst):
                run = (gst[0] == 0) & (g >= g_resume) & \
                    (i_load + g * GRP >= i)

                def proc(gst):
                    prev_id0 = gst[1]
                    base = g * GRP
                    cvec = cl_v[pl.ds(base, GRP)]
                    pxv = xs_v[pl.ds(base, GRP)]
                    pyv = ys_v[pl.ds(base, GRP)]
                    pzv = zs_v[pl.ds(base, GRP)]
                    c0 = cvec[0]
                    c15 = cvec[GRP - 1]
                    uniform = (c0 == c15) & (c0 >= stg_base) & (c0 < lim)

                    def fast(_):
                        chg = c0 != prev_id0
                        acc = [jnp.where(chg, neg_vec,
                                         acc_v[pl.ds(16 * r, 16)])
                               for r in range(NREG)]
                        for j in range(GRP):
                            px = pxv[j]
                            py = pyv[j]
                            pz = pzv[j]
                            for r in range(NREG):
                                acc[r] = jnp.maximum(
                                    acc[r], px * w[0][r] + py * w[1][r]
                                    + pz * w[2][r])
                        row = (c0 - stg_base) * F1
                        for r in range(NREG):
                            acc_v[pl.ds(16 * r, 16)] = acc[r]
                            stg_v[pl.ds(row + 16 * r, 16)] = acc[r]
                        return (jnp.int32(0), c0)

                    def slow(_):
                        prev_id = prev_id0
                        acc = [acc_v[pl.ds(16 * r, 16)] for r in range(NREG)]
                        for j in range(GRP):
                            c_i = cvec[j]
                            act = (c_i >= stg_base) & (c_i < lim)
                            chg = act & (c_i != prev_id)
                            actpen = jnp.broadcast_to(
                                jnp.where(act, 0.0, NEG), (16,))
                            chgpen = jnp.broadcast_to(
                                jnp.where(chg, NEG * 2.0, 0.0), (16,))
                            row = jnp.where(act, c_i - stg_base,
                                            jnp.int32(STG)) * F1
                            px = pxv[j]
                            py = pyv[j]
                            pz = pzv[j]
                            for r in range(NREG):
                                acc[r] = jnp.maximum(
                                    acc[r] + chgpen,
                                    actpen + px * w[0][r] + py * w[1][r]
                                    + pz * w[2][r])
                                stg_v[pl.ds(row + 16 * r, 16)] = acc[r]
                            prev_id = jnp.where(act, c_i, prev_id)
                        for r in range(NREG):
                            acc_v[pl.ds(16 * r, 16)] = acc[r]
                        pause = jnp.minimum(c15, c_last) >= win_end
                        newpaused = jnp.where(pause, g + 1, 0)
                        return (newpaused, prev_id)

                    return lax.cond(uniform, fast, slow, 0)

                return lax.cond(run, proc, lambda s: s, gst)

            gst = lax.fori_loop(0, NGRP, group_fn, (jnp.int32(0), prev_id))
            pausedg = gst[0]
            prev_id = gst[1]
            consumed = pausedg == 0
            blk_last = cl_v[pl.ds(BLK - 16, 16)][15]
            i_new = jnp.where(consumed, i + BLK, i)
            return (i_new,
                    jnp.where(consumed, 0, 1),
                    jnp.where(consumed, 0, 1),
                    jnp.where(consumed, 0, pausedg - 1),
                    jnp.where(consumed & ((blk_last > c_last)
                                          | (i_new >= N)), 1, 0),
                    stg_base, prev_id)

        est2 = lax.cond(can_process, do_process, lambda s: s,
                        (i, block_valid, paused, g_resume, done_data,
                         stg_base, prev_id))
        (i, block_valid, paused, g_resume, done_data, stg_base, prev_id) = est2

        flush = (~done_all) & ((paused == 1)
                               | ((done_data == 1) & (stg_base <= c_last)))

        @pl.when(flush)
        def _():
            def brow(rr, _):
                off = rr * F1
                for j in range(NREG):
                    stg_v[pl.ds(off + 16 * j, 16)] = jnp.maximum(
                        stg_v[pl.ds(off + 16 * j, 16)] + bias[j], 0.0)
                return 0
            lax.fori_loop(0, STG, brow, 0)
            a = jnp.maximum(c_prev + 1 - stg_base, 0)
            b = jnp.minimum(jnp.int32(STG), c_last + 1 - stg_base)
            rem = jnp.maximum(b - a, 0)
            for sz in (128, 64, 32, 16, 8, 4, 2, 1):
                off = a + (rem & (~(2 * sz - 1)))

                @pl.when((rem & sz) != 0)
                def _(off=off, sz=sz):
                    src = pl.multiple_of(off * F1, 128)
                    dst = pl.multiple_of((stg_base + off) * F1, 128)
                    pltpu.sync_copy(stg_v.at[pl.ds(src, sz * F1)],
                                    out_hbm.at[pl.ds(dst, sz * F1)])
            def zrow2(g, _):
                for j in range(GRP):
                    stg_v[pl.ds(g * (GRP * 16) + 16 * j, 16)] = neg_vec
                return 0
            lax.fori_loop(0, (STG * F1) // (GRP * 16), zrow2, 0)
            for r in range(NREG):
                acc_v[pl.ds(16 * r, 16)] = neg_vec

        stg_base = jnp.where(flush, stg_base + STG, stg_base)
        paused = jnp.where(flush, 0, paused)
        return (i, block_valid, paused, g_resume, done_data, stg_base,
                prev_id)

    stg_base0 = ((c_prev + 1) // STG) * STG
    est0 = ((wid * C).astype(jnp.int32), jnp.int32(0), jnp.int32(0),
            jnp.int32(0), jnp.int32(0), stg_base0, jnp.int32(-1))
    lax.fori_loop(0, EMAX, event_body, est0)


_sc_segmax = functools.partial(
    pl.kernel,
    out_type=jax.ShapeDtypeStruct((NUM_SEG * F1,), jnp.float32),
    mesh=plsc.VectorSubcoreMesh(core_axis_name="c", subcore_axis_name="s"),
    scratch_types=[
        pltpu.VMEM((BLK,), jnp.float32),
        pltpu.VMEM((BLK,), jnp.float32),
        pltpu.VMEM((BLK,), jnp.float32),
        pltpu.VMEM((BLK,), jnp.int32),
        pltpu.VMEM((3, F1), jnp.float32),
        pltpu.VMEM((F1,), jnp.float32),
        pltpu.VMEM((NW + 16,), jnp.int32),
        pltpu.VMEM((NW + 16,), jnp.int32),
        pltpu.VMEM(((STG + 1) * F1,), jnp.float32),
        pltpu.VMEM((F1,), jnp.float32),
    ],
)(_sc_body)


_ROWS = 1000


def _mlp_body(x_ref, wg1_ref, bg1_ref, wg2_ref, bg2_ref, out_ref):
    x = x_ref[...]
    h = jnp.maximum(
        jnp.dot(x, wg1_ref[...], preferred_element_type=jnp.float32)
        + bg1_ref[...], 0.0)
    out_ref[...] = jnp.maximum(
        jnp.dot(h, wg2_ref[...], preferred_element_type=jnp.float32)
        + bg2_ref[...], 0.0)


def _mlp(x, Wg1, bg1, Wg2, bg2):
    return pl.pallas_call(
        _mlp_body,
        grid=(NUM_SEG // _ROWS,),
        in_specs=[
            pl.BlockSpec((_ROWS, F1), lambda i: (i, 0)),
            pl.BlockSpec((F1, G1), lambda i: (0, 0)),
            pl.BlockSpec((1, G1), lambda i: (0, 0)),
            pl.BlockSpec((G1, G2), lambda i: (0, 0)),
            pl.BlockSpec((1, G2), lambda i: (0, 0)),
        ],
        out_specs=pl.BlockSpec((_ROWS, G2), lambda i: (i, 0)),
        out_shape=jax.ShapeDtypeStruct((NUM_SEG, G2), jnp.float32),
    )(x, Wg1, bg1.reshape(1, G1), Wg2, bg2.reshape(1, G2))


def kernel(relative_points, cluster, W1, b1, Wg1, bg1, Wg2, bg2):
    pts_t = relative_points.T
    cb = cluster[C - 1::C]
    pad = jnp.zeros((16,), jnp.int32)
    cprev = jnp.concatenate(
        [jnp.full((1,), -1, jnp.int32), cb[:-1], pad])[:NW + 16]
    clast = jnp.concatenate([cb.at[NW - 1].set(NUM_SEG - 1), pad])
    mx = _sc_segmax(pts_t[0], pts_t[1], pts_t[2], cluster, W1, b1,
                    cprev, clast).reshape(NUM_SEG, F1)
    return _mlp(mx, Wg1, bg1, Wg2, bg2)


# speedup vs baseline: 1.0034x; 1.0034x over previous
import functools

import jax
import jax.numpy as jnp
from jax import lax
from jax.experimental import pallas as pl
from jax.experimental.pallas import tpu as pltpu
from jax.experimental.pallas import tpu_sc as plsc

N = 320000
NUM_SEG = 10000
F1 = 128
G1 = 256
G2 = 128

NW = 32
C = N // NW
BLK = 1024
GRP = 16
NGRP = BLK // GRP
STG = 128
NEG = -3.0e38
NREG = F1 // 16
EMAX = N // BLK + NUM_SEG // STG + 4


def _sc_body(xs_hbm, ys_hbm, zs_hbm, cl_hbm, w1_hbm, b1_hbm,
             cprev_hbm, clast_hbm, out_hbm,
             xs_v, ys_v, zs_v, cl_v, w1_v, b1_v, cprev_v, clast_v, stg_v,
             acc_v):
    wid = lax.axis_index("s") * 2 + lax.axis_index("c")
    pltpu.sync_copy(w1_hbm, w1_v)
    pltpu.sync_copy(b1_hbm, b1_v)
    pltpu.sync_copy(cprev_hbm, cprev_v)
    pltpu.sync_copy(clast_hbm, clast_v)

    c_prev = cprev_v[pl.ds(wid, 16)][0]
    c_last = clast_v[pl.ds(wid, 16)][0]

    w = [[w1_v[r, pl.ds(16 * j, 16)] for j in range(NREG)] for r in range(3)]
    bias = [b1_v[pl.ds(16 * j, 16)] for j in range(NREG)]
    neg_vec = jnp.full((16,), NEG, jnp.float32)
    zero_vec = jnp.zeros((16,), jnp.float32)

    def zrow(g, _):
        for j in range(GRP):
            stg_v[pl.ds(g * (GRP * 16) + 16 * j, 16)] = neg_vec
        return 0
    lax.fori_loop(0, (STG * F1) // (GRP * 16), zrow, 0)
    for r in range(NREG):
        acc_v[pl.ds(16 * r, 16)] = neg_vec

    def event_body(_, est):
        (i, block_valid, paused, g_resume, done_data, stg_base, prev_id) = est
        done_all = (done_data == 1) & (stg_base > c_last)
        i_load = pl.multiple_of(jnp.minimum(i, N - BLK), 8)
        win_end = stg_base + STG
        lim = jnp.minimum(win_end, c_last + 1)

        need_load = (~done_all) & (done_data == 0) & (block_valid == 0)

        @pl.when(need_load)
        def _():
            pltpu.sync_copy(cl_hbm.at[pl.ds(i_load, BLK)], cl_v)
            pltpu.sync_copy(xs_hbm.at[pl.ds(i_load, BLK)], xs_v)
            pltpu.sync_copy(ys_hbm.at[pl.ds(i_load, BLK)], ys_v)
            pltpu.sync_copy(zs_hbm.at[pl.ds(i_load, BLK)], zs_v)

        block_valid = jnp.where(need_load, 1, block_valid)
        can_process = (~done_all) & (done_data == 0) & (block_valid == 1)

        def do_process(pst):
            (i, block_valid, paused, g_resume, done_data, stg_base,
             prev_id) = pst

            def group_fn(g, gst):
                run = (gst[0] == 0) & (g >= g_resume) & \
                    (i_load + g * GRP >= i)

                def proc(gst):
                    prev_id0 = gst[1]
                    base = g * GRP
                    cvec = cl_v[pl.ds(base, GRP)]
                    pxv = xs_v[pl.ds(base, GRP)]
                    pyv = ys_v[pl.ds(base, GRP)]
                    pzv = zs_v[pl.ds(base, GRP)]
                    c0 = cvec[0]
                    c15 = cvec[GRP - 1]
                    uniform = (c0 == c15) & (c0 >= stg_base) & (c0 < lim)

                    def fast(_):
                        chg = c0 != prev_id0
                        acc = [jnp.where(chg, neg_vec,
                                         acc_v[pl.ds(16 * r, 16)])
                               for r in range(NREG)]
                        for j in range(GRP):
                            px = pxv[j]
                            py = pyv[j]
                            pz = pzv[j]
                            for r in range(NREG):
                                acc[r] = jnp.maximum(
                                    acc[r], px * w[0][r] + py * w[1][r]
                                    + pz * w[2][r])
                        row = (c0 - stg_base) * F1
                        for r in range(NREG):
                            acc_v[pl.ds(16 * r, 16)] = acc[r]
                            stg_v[pl.ds(row + 16 * r, 16)] = acc[r]
                        return (jnp.int32(0), c0)

                    def slow(_):
                        prev_id = prev_id0
                        acc = [acc_v[pl.ds(16 * r, 16)] for r in range(NREG)]
                        for j in range(GRP):
                            c_i = cvec[j]
                            act = (c_i >= stg_base) & (c_i < lim)
                            chg = act & (c_i != prev_id)
                            chgpen = jnp.broadcast_to(
                                jnp.where(chg, NEG * 2.0, 0.0), (16,))
                            row = jnp.where(act, c_i - stg_base,
                                            jnp.int32(STG)) * F1
                            px = pxv[j]
                            py = pyv[j]
                            pz = pzv[j]
                            for r in range(NREG):
                                acc[r] = jnp.maximum(
                                    acc[r] + chgpen,
                                    px * w[0][r] + py * w[1][r]
                                    + pz * w[2][r])
                                stg_v[pl.ds(row + 16 * r, 16)] = acc[r]
                            prev_id = jnp.where(act, c_i, prev_id)
                        for r in range(NREG):
                            acc_v[pl.ds(16 * r, 16)] = acc[r]
                        pause = jnp.minimum(c15, c_last) >= win_end
                        newpaused = jnp.where(pause, g + 1, 0)
                        return (newpaused, prev_id)

                    return lax.cond(uniform, fast, slow, 0)

                return lax.cond(run, proc, lambda s: s, gst)

            gst = lax.fori_loop(0, NGRP, group_fn, (jnp.int32(0), prev_id))
            pausedg = gst[0]
            prev_id = gst[1]
            consumed = pausedg == 0
            blk_last = cl_v[pl.ds(BLK - 16, 16)][15]
            i_new = jnp.where(consumed, i + BLK, i)
            return (i_new,
                    jnp.where(consumed, 0, 1),
                    jnp.where(consumed, 0, 1),
                    jnp.where(consumed, 0, pausedg - 1),
                    jnp.where(consumed & ((blk_last > c_last)
                                          | (i_new >= N)), 1, 0),
                    stg_base, prev_id)

        est2 = lax.cond(can_process, do_process, lambda s: s,
                        (i, block_valid, paused, g_resume, done_data,
                         stg_base, prev_id))
        (i, block_valid, paused, g_resume, done_data, stg_base, prev_id) = est2

        flush = (~done_all) & ((paused == 1)
                               | ((done_data == 1) & (stg_base <= c_last)))

        @pl.when(flush)
        def _():
            def brow(rr, _):
                off = rr * F1
                for j in range(NREG):
                    stg_v[pl.ds(off + 16 * j, 16)] = jnp.maximum(
                        stg_v[pl.ds(off + 16 * j, 16)] + bias[j], 0.0)
                return 0
            lax.fori_loop(0, STG, brow, 0)
            a = jnp.maximum(c_prev + 1 - stg_base, 0)
            b = jnp.minimum(jnp.int32(STG), c_last + 1 - stg_base)
            rem = jnp.maximum(b - a, 0)
            for sz in (128, 64, 32, 16, 8, 4, 2, 1):
                off = a + (rem & (~(2 * sz - 1)))

                @pl.when((rem & sz) != 0)
                def _(off=off, sz=sz):
                    src = pl.multiple_of(off * F1, 128)
                    dst = pl.multiple_of((stg_base + off) * F1, 128)
                    pltpu.sync_copy(stg_v.at[pl.ds(src, sz * F1)],
                                    out_hbm.at[pl.ds(dst, sz * F1)])
            def zrow2(g, _):
                for j in range(GRP):
                    stg_v[pl.ds(g * (GRP * 16) + 16 * j, 16)] = neg_vec
                return 0
            lax.fori_loop(0, (STG * F1) // (GRP * 16), zrow2, 0)
            for r in range(NREG):
                acc_v[pl.ds(16 * r, 16)] = neg_vec

        stg_base = jnp.where(flush, stg_base + STG, stg_base)
        paused = jnp.where(flush, 0, paused)
        return (i, block_valid, paused, g_resume, done_data, stg_base,
                prev_id)

    stg_base0 = ((c_prev + 1) // STG) * STG
    est0 = ((wid * C).astype(jnp.int32), jnp.int32(0), jnp.int32(0),
            jnp.int32(0), jnp.int32(0), stg_base0, jnp.int32(-1))
    lax.fori_loop(0, EMAX, event_body, est0)


_sc_segmax = functools.partial(
    pl.kernel,
    out_type=jax.ShapeDtypeStruct((NUM_SEG * F1,), jnp.float32),
    mesh=plsc.VectorSubcoreMesh(core_axis_name="c", subcore_axis_name="s"),
    scratch_types=[
        pltpu.VMEM((BLK,), jnp.float32),
        pltpu.VMEM((BLK,), jnp.float32),
        pltpu.VMEM((BLK,), jnp.float32),
        pltpu.VMEM((BLK,), jnp.int32),
        pltpu.VMEM((3, F1), jnp.float32),
        pltpu.VMEM((F1,), jnp.float32),
        pltpu.VMEM((NW + 16,), jnp.int32),
        pltpu.VMEM((NW + 16,), jnp.int32),
        pltpu.VMEM(((STG + 1) * F1,), jnp.float32),
        pltpu.VMEM((F1,), jnp.float32),
    ],
)(_sc_body)


_ROWS = 1000


def _mlp_body(x_ref, wg1_ref, bg1_ref, wg2_ref, bg2_ref, out_ref):
    x = x_ref[...]
    h = jnp.maximum(
        jnp.dot(x, wg1_ref[...], preferred_element_type=jnp.float32)
        + bg1_ref[...], 0.0)
    out_ref[...] = jnp.maximum(
        jnp.dot(h, wg2_ref[...], preferred_element_type=jnp.float32)
        + bg2_ref[...], 0.0)


def _mlp(x, Wg1, bg1, Wg2, bg2):
    return pl.pallas_call(
        _mlp_body,
        grid=(NUM_SEG // _ROWS,),
        in_specs=[
            pl.BlockSpec((_ROWS, F1), lambda i: (i, 0)),
            pl.BlockSpec((F1, G1), lambda i: (0, 0)),
            pl.BlockSpec((1, G1), lambda i: (0, 0)),
            pl.BlockSpec((G1, G2), lambda i: (0, 0)),
            pl.BlockSpec((1, G2), lambda i: (0, 0)),
        ],
        out_specs=pl.BlockSpec((_ROWS, G2), lambda i: (i, 0)),
        out_shape=jax.ShapeDtypeStruct((NUM_SEG, G2), jnp.float32),
    )(x, Wg1, bg1.reshape(1, G1), Wg2, bg2.reshape(1, G2))


def kernel(relative_points, cluster, W1, b1, Wg1, bg1, Wg2, bg2):
    pts_t = relative_points.T
    cb = cluster[C - 1::C]
    pad = jnp.zeros((16,), jnp.int32)
    cprev = jnp.concatenate(
        [jnp.full((1,), -1, jnp.int32), cb[:-1], pad])[:NW + 16]
    clast = jnp.concatenate([cb.at[NW - 1].set(NUM_SEG - 1), pad])
    mx = _sc_segmax(pts_t[0], pts_t[1], pts_t[2], cluster, W1, b1,
                    cprev, clast).reshape(NUM_SEG, F1)
    return _mlp(mx, Wg1, bg1, Wg2, bg2)


# speedup vs baseline: 1.0673x; 1.0637x over previous
import functools

import jax
import jax.numpy as jnp
from jax import lax
from jax.experimental import pallas as pl
from jax.experimental.pallas import tpu as pltpu
from jax.experimental.pallas import tpu_sc as plsc

N = 320000
NUM_SEG = 10000
F1 = 128
G1 = 256
G2 = 128

NW = 32
C = N // NW
BLK = 1024
GRP = 16
NGRP = BLK // GRP
STG = 128
NEG = -3.0e38
NREG = F1 // 16
EMAX = N // BLK + NUM_SEG // STG + 4


def _sc_body(xs_hbm, ys_hbm, zs_hbm, cl_hbm, w1_hbm, b1_hbm,
             cprev_hbm, clast_hbm, out_hbm,
             xs_v, ys_v, zs_v, cl_v, w1_v, b1_v, cprev_v, clast_v, stg_v,
             acc_v, dsem):
    wid = lax.axis_index("s") * 2 + lax.axis_index("c")
    pltpu.sync_copy(w1_hbm, w1_v)
    pltpu.sync_copy(b1_hbm, b1_v)
    pltpu.sync_copy(cprev_hbm, cprev_v)
    pltpu.sync_copy(clast_hbm, clast_v)

    c_prev = cprev_v[pl.ds(wid, 16)][0]
    c_last = clast_v[pl.ds(wid, 16)][0]

    w = [[w1_v[r, pl.ds(16 * j, 16)] for j in range(NREG)] for r in range(3)]
    bias = [b1_v[pl.ds(16 * j, 16)] for j in range(NREG)]
    neg_vec = jnp.full((16,), NEG, jnp.float32)

    def zrow(g, _):
        for j in range(GRP):
            stg_v[pl.ds(g * (GRP * 16) + 16 * j, 16)] = neg_vec
        return 0
    lax.fori_loop(0, (STG * F1) // (GRP * 16), zrow, 0)
    for r in range(NREG):
        acc_v[pl.ds(16 * r, 16)] = neg_vec

    def event_body(_, est):
        (i, block_valid, paused, g_resume, done_data, stg_base, prev_id) = est
        done_all = (done_data == 1) & (stg_base > c_last)
        i_load = pl.multiple_of(jnp.minimum(i, N - BLK), 8)
        win_end = stg_base + STG
        lim = jnp.minimum(win_end, c_last + 1)

        need_load = (~done_all) & (done_data == 0) & (block_valid == 0)

        @pl.when(need_load)
        def _():
            cp1 = pltpu.async_copy(cl_hbm.at[pl.ds(i_load, BLK)], cl_v, dsem)
            cp2 = pltpu.async_copy(xs_hbm.at[pl.ds(i_load, BLK)], xs_v, dsem)
            cp3 = pltpu.async_copy(ys_hbm.at[pl.ds(i_load, BLK)], ys_v, dsem)
            cp4 = pltpu.async_copy(zs_hbm.at[pl.ds(i_load, BLK)], zs_v, dsem)
            cp1.wait()
            cp2.wait()
            cp3.wait()
            cp4.wait()

        block_valid = jnp.where(need_load, 1, block_valid)
        can_process = (~done_all) & (done_data == 0) & (block_valid == 1)

        def do_process(pst):
            (i, block_valid, paused, g_resume, done_data, stg_base,
             prev_id) = pst

            def group_fn(g, gst):
                run = (gst[0] == 0) & (g >= g_resume) & \
                    (i_load + g * GRP >= i)

                def proc(gst):
                    prev_id0 = gst[1]
                    base = g * GRP
                    cvec = cl_v[pl.ds(base, GRP)]
                    pxv = xs_v[pl.ds(base, GRP)]
                    pyv = ys_v[pl.ds(base, GRP)]
                    pzv = zs_v[pl.ds(base, GRP)]
                    c0 = cvec[0]
                    c15 = cvec[GRP - 1]
                    uniform = (c0 == c15) & (c0 >= stg_base) & (c0 < lim)

                    def fast(_):
                        chg = c0 != prev_id0
                        acc = [jnp.where(chg, neg_vec,
                                         acc_v[pl.ds(16 * r, 16)])
                               for r in range(NREG)]
                        for j in range(GRP):
                            px = pxv[j]
                            py = pyv[j]
                            pz = pzv[j]
                            for r in range(NREG):
                                acc[r] = jnp.maximum(
                                    acc[r], px * w[0][r] + py * w[1][r]
                                    + pz * w[2][r])
                        row = (c0 - stg_base) * F1
                        for r in range(NREG):
                            acc_v[pl.ds(16 * r, 16)] = acc[r]
                            stg_v[pl.ds(row + 16 * r, 16)] = acc[r]
                        return (jnp.int32(0), c0)

                    def slow(_):
                        prev_id = prev_id0
                        acc = [acc_v[pl.ds(16 * r, 16)] for r in range(NREG)]
                        for j in range(GRP):
                            c_i = cvec[j]
                            act = (c_i >= stg_base) & (c_i < lim)
                            chg = act & (c_i != prev_id)
                            chgpen = jnp.broadcast_to(
                                jnp.where(chg, NEG * 2.0, 0.0), (16,))
                            row = jnp.where(act, c_i - stg_base,
                                            jnp.int32(STG)) * F1
                            px = pxv[j]
                            py = pyv[j]
                            pz = pzv[j]
                            for r in range(NREG):
                                acc[r] = jnp.maximum(
                                    acc[r] + chgpen,
                                    px * w[0][r] + py * w[1][r]
                                    + pz * w[2][r])
                                stg_v[pl.ds(row + 16 * r, 16)] = acc[r]
                            prev_id = jnp.where(act, c_i, prev_id)
                        for r in range(NREG):
                            acc_v[pl.ds(16 * r, 16)] = acc[r]
                        pause = jnp.minimum(c15, c_last) >= win_end
                        newpaused = jnp.where(pause, g + 1, 0)
                        return (newpaused, prev_id)

                    return lax.cond(uniform, fast, slow, 0)

                return lax.cond(run, proc, lambda s: s, gst)

            gst = lax.fori_loop(0, NGRP, group_fn, (jnp.int32(0), prev_id))
            pausedg = gst[0]
            prev_id = gst[1]
            consumed = pausedg == 0
            blk_last = cl_v[pl.ds(BLK - 16, 16)][15]
            i_new = jnp.where(consumed, i + BLK, i)
            return (i_new,
                    jnp.where(consumed, 0, 1),
                    jnp.where(consumed, 0, 1),
                    jnp.where(consumed, 0, pausedg - 1),
                    jnp.where(consumed & ((blk_last > c_last)
                                          | (i_new >= N)), 1, 0),
                    stg_base, prev_id)

        est2 = lax.cond(can_process, do_process, lambda s: s,
                        (i, block_valid, paused, g_resume, done_data,
                         stg_base, prev_id))
        (i, block_valid, paused, g_resume, done_data, stg_base, prev_id) = est2

        flush = (~done_all) & ((paused == 1)
                               | ((done_data == 1) & (stg_base <= c_last)))

        @pl.when(flush)
        def _():
            def brow(rr, _):
                off = rr * F1
                for j in range(NREG):
                    stg_v[pl.ds(off + 16 * j, 16)] = jnp.maximum(
                        stg_v[pl.ds(off + 16 * j, 16)] + bias[j], 0.0)
                return 0
            lax.fori_loop(0, STG, brow, 0)
            a = jnp.maximum(c_prev + 1 - stg_base, 0)
            b = jnp.minimum(jnp.int32(STG), c_last + 1 - stg_base)
            rem = jnp.maximum(b - a, 0)
            for sz in (128, 64, 32, 16, 8, 4, 2, 1):
                off = a + (rem & (~(2 * sz - 1)))

                @pl.when((rem & sz) != 0)
                def _(off=off, sz=sz):
                    src = pl.multiple_of(off * F1, 128)
                    dst = pl.multiple_of((stg_base + off) * F1, 128)
                    pltpu.sync_copy(stg_v.at[pl.ds(src, sz * F1)],
                                    out_hbm.at[pl.ds(dst, sz * F1)])
            def zrow2(g, _):
                for j in range(GRP):
                    stg_v[pl.ds(g * (GRP * 16) + 16 * j, 16)] = neg_vec
                return 0
            lax.fori_loop(0, (STG * F1) // (GRP * 16), zrow2, 0)
            for r in range(NREG):
                acc_v[pl.ds(16 * r, 16)] = neg_vec

        stg_base = jnp.where(flush, stg_base + STG, stg_base)
        paused = jnp.where(flush, 0, paused)
        return (i, block_valid, paused, g_resume, done_data, stg_base,
                prev_id)

    stg_base0 = ((c_prev + 1) // STG) * STG
    est0 = ((wid * C).astype(jnp.int32), jnp.int32(0), jnp.int32(0),
            jnp.int32(0), jnp.int32(0), stg_base0, jnp.int32(-1))
    lax.fori_loop(0, EMAX, event_body, est0)


_sc_segmax = functools.partial(
    pl.kernel,
    out_type=jax.ShapeDtypeStruct((NUM_SEG * F1,), jnp.float32),
    mesh=plsc.VectorSubcoreMesh(core_axis_name="c", subcore_axis_name="s"),
    scratch_types=[
        pltpu.VMEM((BLK,), jnp.float32),
        pltpu.VMEM((BLK,), jnp.float32),
        pltpu.VMEM((BLK,), jnp.float32),
        pltpu.VMEM((BLK,), jnp.int32),
        pltpu.VMEM((3, F1), jnp.float32),
        pltpu.VMEM((F1,), jnp.float32),
        pltpu.VMEM((NW + 16,), jnp.int32),
        pltpu.VMEM((NW + 16,), jnp.int32),
        pltpu.VMEM(((STG + 1) * F1,), jnp.float32),
        pltpu.VMEM((F1,), jnp.float32),
        pltpu.SemaphoreType.DMA,
    ],
)(_sc_body)


_ROWS = 2000


def _mlp_body(x_ref, wg1_ref, bg1_ref, wg2_ref, bg2_ref, out_ref):
    x = x_ref[...]
    h = jnp.maximum(
        jnp.dot(x, wg1_ref[...], preferred_element_type=jnp.float32)
        + bg1_ref[...], 0.0)
    out_ref[...] = jnp.maximum(
        jnp.dot(h, wg2_ref[...], preferred_element_type=jnp.float32)
        + bg2_ref[...], 0.0)


def _mlp(x, Wg1, bg1, Wg2, bg2):
    return pl.pallas_call(
        _mlp_body,
        grid=(NUM_SEG // _ROWS,),
        in_specs=[
            pl.BlockSpec((_ROWS, F1), lambda i: (i, 0)),
            pl.BlockSpec((F1, G1), lambda i: (0, 0)),
            pl.BlockSpec((1, G1), lambda i: (0, 0)),
            pl.BlockSpec((G1, G2), lambda i: (0, 0)),
            pl.BlockSpec((1, G2), lambda i: (0, 0)),
        ],
        out_specs=pl.BlockSpec((_ROWS, G2), lambda i: (i, 0)),
        out_shape=jax.ShapeDtypeStruct((NUM_SEG, G2), jnp.float32),
    )(x, Wg1, bg1.reshape(1, G1), Wg2, bg2.reshape(1, G2))


def kernel(relative_points, cluster, W1, b1, Wg1, bg1, Wg2, bg2):
    pts_t = relative_points.T
    cb = cluster[C - 1::C]
    pad = jnp.zeros((16,), jnp.int32)
    cprev = jnp.concatenate(
        [jnp.full((1,), -1, jnp.int32), cb[:-1], pad])[:NW + 16]
    clast = jnp.concatenate([cb.at[NW - 1].set(NUM_SEG - 1), pad])
    mx = _sc_segmax(pts_t[0], pts_t[1], pts_t[2], cluster, W1, b1,
                    cprev, clast).reshape(NUM_SEG, F1)
    return _mlp(mx, Wg1, bg1, Wg2, bg2)
